# baseline (speedup 1.0000x reference)
import functools

import jax
import jax.numpy as jnp
from jax import lax
from jax.experimental import pallas as pl
from jax.experimental.pallas import tpu as pltpu
from jax.experimental.pallas import tpu_sc as plsc

DIM = 64
B = 16384
L2_REG = 1e-4
V = 1000000

NC = 2
NS = 16
L = 16
NW = NC * NS
BPW = B // NW

NBLK_PAD = 8192
NCH = 128
CAP_U = 256
CAP_I = 512
TAIL = 128
TAIL_BASE = 999936
LAST_COL = 999808


def _prep(ids, n_out, cap):
    n = ids.shape[0]
    blk = ids >> 7
    order = jnp.argsort(blk).astype(jnp.int32)
    sid = jnp.take(ids, order)
    sblk = sid >> 7
    starts = jnp.searchsorted(
        sblk, jnp.arange(NBLK_PAD + 1, dtype=sblk.dtype)
    ).astype(jnp.int32)
    cs = starts[::64]
    kar = jnp.arange(cap, dtype=jnp.int32)
    g = cs[:NCH, None] + kar[None, :]
    valid = g < cs[1 : NCH + 1, None]
    gc = jnp.minimum(g, n - 1)
    sid_p = jnp.where(valid, jnp.take(sid, gc), 0).astype(jnp.int32)
    dst_p = jnp.where(valid, jnp.take(order, gc), n_out).astype(jnp.int32)
    st2 = starts[:NBLK_PAD].reshape(NCH, 64)
    loc = jnp.concatenate(
        [st2, cs[1 : NCH + 1, None], jnp.zeros((NCH, 63), jnp.int32)], axis=1
    )
    loc = loc - cs[:NCH, None]
    lim = starts[7812] - cs[:NCH]
    loc = jnp.clip(jnp.minimum(loc, lim[:, None]), 0, cap)
    tmask = ids >= TAIL_BASE
    tcnt = jnp.sum(tmask.astype(jnp.int32))
    tord = jnp.argsort(~tmask).astype(jnp.int32)[:TAIL]
    tval = jnp.arange(TAIL, dtype=jnp.int32) < tcnt
    t_sid = jnp.where(tval, jnp.take(ids, tord), TAIL_BASE).astype(jnp.int32)
    t_dst = jnp.where(tval, tord, n_out).astype(jnp.int32)
    return (
        sid_p,
        dst_p.reshape(NCH, cap // 128, 128),
        loc,
        t_sid,
        t_dst.reshape(1, TAIL),
        tcnt.astype(jnp.int32),
    )


def _sc_sweep(tu, ti, u_meta, i_meta):
    u_sid, u_dst, u_loc, u_tsid, u_tdst, u_tcnt = u_meta
    i_sid, i_dst, i_loc, i_tsid, i_tdst, i_tcnt = i_meta
    tmeta = jnp.stack([u_tcnt, i_tcnt]).reshape(2)
    tmeta = jnp.concatenate([tmeta, jnp.zeros((126,), jnp.int32)])

    mesh = plsc.VectorSubcoreMesh(core_axis_name="c", subcore_axis_name="s")

    @functools.partial(
        pl.kernel,
        mesh=mesh,
        compiler_params=pltpu.CompilerParams(needs_layout_passes=False),
        out_type=(
            jax.ShapeDtypeStruct((B + 8, 128), jnp.float32),
            jax.ShapeDtypeStruct((2 * B + 8, 128), jnp.float32),
        ),
        scratch_types=[
            pltpu.VMEM((CAP_I + 16,), jnp.int32),
            pltpu.VMEM((144,), jnp.int32),
            pltpu.VMEM((4, 128), jnp.int32),
            pltpu.VMEM((128,), jnp.int32),
            pltpu.VMEM((DIM, 128), jnp.float32),
            pltpu.VMEM((DIM, 128), jnp.float32),
            pltpu.VMEM((DIM, 128), jnp.float32),
            pltpu.VMEM((DIM, 128), jnp.float32),
            pltpu.VMEM((DIM, 64), jnp.float32),
            pltpu.VMEM((CAP_I, 128), jnp.float32),
            pltpu.SemaphoreType.DMA,
            pltpu.SemaphoreType.DMA,
            pltpu.SemaphoreType.DMA,
            pltpu.SemaphoreType.DMA,
            pltpu.SemaphoreType.DMA,
        ],
    )
    def k(tu_h, ti_h, usid_h, udst_h, uloc_h, utsid_h, utdst_h,
          isid_h, idst_h, iloc_h, itsid_h, itdst_h, tmeta_h,
          ou, oi,
          sidv, locv, dstv, tmv, b0, b1, b2, b3, tblk, stage,
          s0, s1, s2, s3, ssc):
        core = lax.axis_index("c")
        sub = lax.axis_index("s")
        bufs = [b0, b1, b2, b3]
        sems = [s0, s1, s2, s3]
        lanes = lax.iota(jnp.int32, L)

        def sweep(t_h, sid_h, dst_h, loc_h, out_h, cap):
            nreg = cap // 128

            def chunk(cc, _):
                c = cc * NS + sub
                pltpu.sync_copy(sid_h.at[c], sidv.at[pl.ds(0, cap)])
                pltpu.sync_copy(loc_h.at[c], locv.at[pl.ds(0, 128)])
                pltpu.sync_copy(dst_h.at[c], dstv.at[pl.ds(0, nreg), :])

                def col_of(b):
                    return jnp.minimum(b * 128, LAST_COL)

                base_b = c * 64

                def issue(boff, bi):
                    pltpu.async_copy(
                        t_h.at[:, pl.ds(col_of(base_b + boff), 128)],
                        bufs[bi],
                        sems[bi],
                    )

                def wait_buf(bi):
                    pltpu.make_async_copy(
                        t_h.at[:, pl.ds(0, 128)], bufs[bi], sems[bi]
                    ).wait()

                def process(joff, bi):
                    bcol = col_of(base_b + joff)
                    lo = locv[pl.ds(joff, 16)][0]
                    hi = locv[pl.ds(joff + 1, 16)][0]

                    def hit(kk, _):
                        col = sidv[pl.ds(kk, 16)][0] - bcol
                        cv = jnp.full((L,), 0, jnp.int32) + col
                        for k4 in range(4):
                            v = plsc.load_gather(
                                bufs[bi], [lanes + L * k4, cv]
                            )
                            stage[kk, pl.ds(L * k4, L)] = v
                        return 0

                    lax.fori_loop(lo, hi, hit, 0)

                issue(0, 0)
                issue(1, 1)

                def quad(q, _):
                    q4 = q * 4
                    issue(q4 + 2, 2)
                    issue(q4 + 3, 3)
                    wait_buf(0)
                    process(q4 + 0, 0)
                    wait_buf(1)
                    process(q4 + 1, 1)
                    issue(q4 + 4, 0)
                    issue(q4 + 5, 1)
                    wait_buf(2)
                    process(q4 + 2, 2)
                    wait_buf(3)
                    process(q4 + 3, 3)
                    return 0

                lax.fori_loop(0, 16, quad, 0)
                wait_buf(0)
                wait_buf(1)
                for m in range(nreg):
                    pltpu.async_copy(
                        stage.at[pl.ds(m * 128, 128), :],
                        out_h.at[dstv.at[m]],
                        ssc,
                    ).wait()
                return 0

            lax.fori_loop(0, 8, chunk, 0)

        def tail(t_h, tsid_h, tdst_h, out_h, which):
            pltpu.sync_copy(tsid_h.at[pl.ds(0, TAIL)], sidv.at[pl.ds(0, TAIL)])
            pltpu.sync_copy(tdst_h.at[0], dstv.at[0])
            pltpu.sync_copy(tmeta_h.at[pl.ds(0, 128)], tmv)
            pltpu.async_copy(t_h.at[:, pl.ds(TAIL_BASE, 64)], tblk, s0).wait()
            cnt = tmv[pl.ds(0, 16)][which]

            def hit(kk, _):
                col = sidv[pl.ds(kk, 16)][0] - TAIL_BASE
                cv = jnp.full((L,), 0, jnp.int32) + col
                for k4 in range(4):
                    v = plsc.load_gather(tblk, [lanes + L * k4, cv])
                    stage[kk, pl.ds(L * k4, L)] = v
                return 0

            lax.fori_loop(0, cnt, hit, 0)
            pltpu.async_copy(
                stage.at[pl.ds(0, 128), :], out_h.at[dstv.at[0]], ssc
            ).wait()

        @pl.when(core == 0)
        def _():
            sweep(tu_h, usid_h, udst_h, uloc_h, ou, CAP_U)

            @pl.when(sub == 0)
            def _():
                tail(tu_h, utsid_h, utdst_h, ou, 0)

        @pl.when(core == 1)
        def _():
            sweep(ti_h, isid_h, idst_h, iloc_h, oi, CAP_I)

            @pl.when(sub == 0)
            def _():
                tail(ti_h, itsid_h, itdst_h, oi, 1)

    return k(tu, ti, u_sid, u_dst, u_loc, u_tsid, u_tdst,
             i_sid, i_dst, i_loc, i_tsid, i_tdst, tmeta)


def _sc_scores(ou, oi):
    mesh = plsc.VectorSubcoreMesh(core_axis_name="c", subcore_axis_name="s")
    HALF = BPW // 2
    HG = HALF // L

    @functools.partial(
        pl.kernel,
        mesh=mesh,
        compiler_params=pltpu.CompilerParams(needs_layout_passes=False),
        out_type=(
            jax.ShapeDtypeStruct((B,), jnp.float32),
            jax.ShapeDtypeStruct((B,), jnp.float32),
            jax.ShapeDtypeStruct((NW, L), jnp.float32),
        ),
        scratch_types=[
            pltpu.VMEM((HALF, 128), jnp.float32),
            pltpu.VMEM((HALF, 128), jnp.float32),
            pltpu.VMEM((HALF, 128), jnp.float32),
            pltpu.VMEM((BPW,), jnp.float32),
            pltpu.VMEM((BPW,), jnp.float32),
            pltpu.VMEM((L,), jnp.float32),
            pltpu.VMEM((L * L,), jnp.float32),
            pltpu.VMEM((L * L,), jnp.float32),
            pltpu.SemaphoreType.DMA,
            pltpu.SemaphoreType.DMA,
            pltpu.SemaphoreType.DMA,
        ],
    )
    def k(ou_h, oi_h, pos_out, neg_out, sq_out,
          ubuf, pbuf, nbuf, psc, nsc, sqv, tpm, tnm, su, sp, sn):
        wid = lax.axis_index("s") * NC + lax.axis_index("c")
        base = wid * BPW
        lanes = lax.iota(jnp.int32, L)

        def half(h, sq):
            hb = pl.multiple_of(h * (BPW // 2), BPW // 2)
            cu = pltpu.async_copy(
                ou_h.at[pl.ds(base + hb, HALF), :], ubuf, su)
            cp = pltpu.async_copy(
                oi_h.at[pl.ds(base + hb, HALF), :], pbuf, sp)
            cn = pltpu.async_copy(
                oi_h.at[pl.ds(B + base + hb, HALF), :], nbuf, sn)
            cu.wait()
            cp.wait()
            cn.wait()

            def group(g, sq):
                gbase = pl.multiple_of(g * L, L)
                for r in range(L):
                    tp = jnp.zeros((L,), jnp.float32)
                    tn = jnp.zeros((L,), jnp.float32)
                    sr = jnp.zeros((L,), jnp.float32)
                    for kk in range(DIM // L):
                        u = ubuf[gbase + r, pl.ds(kk * L, L)]
                        p = pbuf[gbase + r, pl.ds(kk * L, L)]
                        n = nbuf[gbase + r, pl.ds(kk * L, L)]
                        tp = tp + u * p
                        tn = tn + u * n
                        sr = sr + (u * u + (p * p + n * n))
                    sq = sq + sr
                    colidx = lanes * L + r
                    plsc.store_scatter(tpm, [colidx], tp)
                    plsc.store_scatter(tnm, [colidx], tn)
                pos_v = jnp.zeros((L,), jnp.float32)
                neg_v = jnp.zeros((L,), jnp.float32)
                for l in range(L):
                    pos_v = pos_v + tpm[pl.ds(l * L, L)]
                    neg_v = neg_v + tnm[pl.ds(l * L, L)]
                psc[pl.ds(hb + gbase, L)] = pos_v
                nsc[pl.ds(hb + gbase, L)] = neg_v
                return sq

            return lax.fori_loop(0, HG, group, sq)

        sq = lax.fori_loop(0, 2, half, jnp.zeros((L,), jnp.float32))
        sqv[...] = sq
        pltpu.sync_copy(psc, pos_out.at[pl.ds(base, BPW)])
        pltpu.sync_copy(nsc, neg_out.at[pl.ds(base, BPW)])
        pltpu.sync_copy(sqv, sq_out.at[wid])

    return k(ou, oi)


def _tc_finalize(pos2, neg2, sq2):
    def body(p_ref, n_ref, s_ref, bpr_ref, auc_ref, reg_ref):
        p = p_ref[...]
        n = n_ref[...]
        d = n - p
        sp = jnp.maximum(d, 0.0) + jnp.log(1.0 + jnp.exp(-jnp.abs(d)))
        bpr_ref[0, 0] = jnp.sum(sp) * (1.0 / B)
        auc_ref[0, 0] = jnp.sum((p > n).astype(jnp.float32)) * (1.0 / B)
        reg_ref[0, 0] = (0.5 * L2_REG / B) * jnp.sum(s_ref[...])

    return pl.pallas_call(
        body,
        out_shape=(
            jax.ShapeDtypeStruct((1, 1), jnp.float32),
            jax.ShapeDtypeStruct((1, 1), jnp.float32),
            jax.ShapeDtypeStruct((1, 1), jnp.float32),
        ),
        out_specs=(
            pl.BlockSpec(memory_space=pltpu.SMEM),
            pl.BlockSpec(memory_space=pltpu.SMEM),
            pl.BlockSpec(memory_space=pltpu.SMEM),
        ),
    )(pos2, neg2, sq2)


def kernel(user_table, item_table, users_id, pos_items_id, neg_items_id):
    uid = users_id.astype(jnp.int32)
    pid = pos_items_id.astype(jnp.int32)
    nid = neg_items_id.astype(jnp.int32)
    iid = jnp.concatenate([pid, nid])
    u_meta = _prep(uid, B, CAP_U)
    i_meta = _prep(iid, 2 * B, CAP_I)
    ou, oi = _sc_sweep(user_table.T, item_table.T, u_meta, i_meta)
    pos_s, neg_s, sq = _sc_scores(ou, oi)
    bpr, auc, reg = _tc_finalize(
        pos_s.reshape(128, 128), neg_s.reshape(128, 128), sq.reshape(4, 128)
    )
    return (bpr[0, 0], auc[0, 0], reg[0, 0])


# speedup vs baseline: 4.2684x; 4.2684x over previous
import functools

import jax
import jax.numpy as jnp
from jax import lax
from jax.experimental import pallas as pl
from jax.experimental.pallas import tpu as pltpu
from jax.experimental.pallas import tpu_sc as plsc

DIM = 64
B = 16384
L2_REG = 1e-4

NC = 2
NS = 16
L = 16
NW = NC * NS
BPW = B // NW
GROUPS = BPW // L


def _sc_scores(user_table, item_table, users_id, pos_items_id, neg_items_id):
    mesh = plsc.VectorSubcoreMesh(core_axis_name="c", subcore_axis_name="s")

    @functools.partial(
        pl.kernel,
        mesh=mesh,
        compiler_params=pltpu.CompilerParams(
            needs_layout_passes=False, use_tc_tiling_on_sc=False
        ),
        out_type=(
            jax.ShapeDtypeStruct((B,), jnp.float32),
            jax.ShapeDtypeStruct((B,), jnp.float32),
            jax.ShapeDtypeStruct((NW, L), jnp.float32),
        ),
        scratch_types=[
            pltpu.VMEM((BPW,), jnp.int32),
            pltpu.VMEM((BPW,), jnp.int32),
            pltpu.VMEM((BPW,), jnp.int32),
            pltpu.VMEM((BPW, DIM), jnp.float32),
            pltpu.VMEM((BPW, DIM), jnp.float32),
            pltpu.VMEM((BPW, DIM), jnp.float32),
            pltpu.VMEM((BPW,), jnp.float32),
            pltpu.VMEM((BPW,), jnp.float32),
            pltpu.VMEM((L,), jnp.float32),
            pltpu.VMEM((L * L,), jnp.float32),
            pltpu.VMEM((L * L,), jnp.float32),
            pltpu.SemaphoreType.DMA,
            pltpu.SemaphoreType.DMA,
            pltpu.SemaphoreType.DMA,
        ],
    )
    def k(uid_hbm, pid_hbm, nid_hbm, ut_hbm, it_hbm,
          pos_out, neg_out, sq_out,
          uidx, pidx, nidx, urows, prows, nrows, psc, nsc, sqv, tpm, tnm,
          sem_u, sem_p, sem_n):
        wid = lax.axis_index("s") * NC + lax.axis_index("c")
        base = wid * BPW
        pltpu.sync_copy(uid_hbm.at[pl.ds(base, BPW)], uidx)
        pltpu.sync_copy(pid_hbm.at[pl.ds(base, BPW)], pidx)
        pltpu.sync_copy(nid_hbm.at[pl.ds(base, BPW)], nidx)
        cu = pltpu.async_copy(ut_hbm.at[uidx], urows, sem_u)
        cp = pltpu.async_copy(it_hbm.at[pidx], prows, sem_p)
        cn = pltpu.async_copy(it_hbm.at[nidx], nrows, sem_n)
        cu.wait()
        cp.wait()
        cn.wait()

        lanes = lax.iota(jnp.int32, L)

        def group(g, sq):
            gbase = pl.multiple_of(g * L, L)
            for r in range(L):
                tp = jnp.zeros((L,), jnp.float32)
                tn = jnp.zeros((L,), jnp.float32)
                sr = jnp.zeros((L,), jnp.float32)
                for kk in range(DIM // L):
                    u = urows[gbase + r, pl.ds(kk * L, L)]
                    p = prows[gbase + r, pl.ds(kk * L, L)]
                    n = nrows[gbase + r, pl.ds(kk * L, L)]
                    tp = tp + u * p
                    tn = tn + u * n
                    sr = sr + (u * u + (p * p + n * n))
                sq = sq + sr
                colidx = lanes * L + r
                plsc.store_scatter(tpm, [colidx], tp)
                plsc.store_scatter(tnm, [colidx], tn)
            pos_v = jnp.zeros((L,), jnp.float32)
            neg_v = jnp.zeros((L,), jnp.float32)
            for l in range(L):
                pos_v = pos_v + tpm[pl.ds(l * L, L)]
                neg_v = neg_v + tnm[pl.ds(l * L, L)]
            psc[pl.ds(gbase, L)] = pos_v
            nsc[pl.ds(gbase, L)] = neg_v
            return sq

        sq = lax.fori_loop(0, GROUPS, group, jnp.zeros((L,), jnp.float32))
        sqv[...] = sq
        pltpu.sync_copy(psc, pos_out.at[pl.ds(base, BPW)])
        pltpu.sync_copy(nsc, neg_out.at[pl.ds(base, BPW)])
        pltpu.sync_copy(sqv, sq_out.at[wid])

    return k(users_id, pos_items_id, neg_items_id, user_table, item_table)


def _tc_finalize(pos2, neg2, sq2):
    def body(p_ref, n_ref, s_ref, bpr_ref, auc_ref, reg_ref):
        p = p_ref[...]
        n = n_ref[...]
        d = n - p
        sp = jnp.maximum(d, 0.0) + jnp.log(1.0 + jnp.exp(-jnp.abs(d)))
        bpr_ref[0, 0] = jnp.sum(sp) * (1.0 / B)
        auc_ref[0, 0] = jnp.sum((p > n).astype(jnp.float32)) * (1.0 / B)
        reg_ref[0, 0] = (0.5 * L2_REG / B) * jnp.sum(s_ref[...])

    return pl.pallas_call(
        body,
        out_shape=(
            jax.ShapeDtypeStruct((1, 1), jnp.float32),
            jax.ShapeDtypeStruct((1, 1), jnp.float32),
            jax.ShapeDtypeStruct((1, 1), jnp.float32),
        ),
        out_specs=(
            pl.BlockSpec(memory_space=pltpu.SMEM),
            pl.BlockSpec(memory_space=pltpu.SMEM),
            pl.BlockSpec(memory_space=pltpu.SMEM),
        ),
    )(pos2, neg2, sq2)


def kernel(user_table, item_table, users_id, pos_items_id, neg_items_id):
    uid = users_id.astype(jnp.int32)
    pid = pos_items_id.astype(jnp.int32)
    nid = neg_items_id.astype(jnp.int32)
    pos_s, neg_s, sq = _sc_scores(user_table, item_table, uid, pid, nid)
    bpr, auc, reg = _tc_finalize(
        pos_s.reshape(128, 128), neg_s.reshape(128, 128), sq.reshape(4, 128)
    )
    return (bpr[0, 0], auc[0, 0], reg[0, 0])
